# baseline (speedup 1.0000x reference)
import functools

import jax
import jax.numpy as jnp
from jax import lax
from jax.experimental import pallas as pl
from jax.experimental.pallas import tpu as pltpu
from jax.experimental.pallas import tpu_sc as plsc

VOCAB = 1000000
EMB = 32
C = 6
B = 16384
PACK = 4
ROWS = VOCAB // PACK
NC = 2
NS = 16
NW = NC * NS
BPW = B // NW
CPW = BPW * C
NSB = 4
SB = BPW // NSB
SC = SB * C
CHUNK = 128

_mesh = plsc.VectorSubcoreMesh(core_axis_name="c", subcore_axis_name="s")


@functools.partial(
    pl.kernel,
    mesh=_mesh,
    compiler_params=pltpu.CompilerParams(needs_layout_passes=False),
    out_type=jax.ShapeDtypeStruct((B * C,), jnp.float32),
    scratch_types=[
        pltpu.VMEM((BPW,), jnp.int32),
        pltpu.VMEM((CPW,), jnp.int32),
        pltpu.VMEM((SB,), jnp.int32),
        pltpu.VMEM((SC,), jnp.int32),
        pltpu.VMEM((SB,), jnp.int32),
        pltpu.VMEM((SC,), jnp.int32),
        pltpu.VMEM((SB, 128), jnp.float32),
        pltpu.VMEM((SC, 128), jnp.float32),
        pltpu.VMEM((CPW,), jnp.float32),
        pltpu.SemaphoreType.DMA,
    ],
)
def _w2v(tgt_hbm, ctx_hbm, ttab_hbm, ctab_hbm, out_hbm,
         tidx_v, cidx_v, tg_v, cg_v, tq_v, cq_v, trow_v, crow_v, out_v, sem):
    wid = lax.axis_index("s") * NC + lax.axis_index("c")
    tbase = wid * BPW
    cbase = wid * CPW

    pltpu.sync_copy(tgt_hbm.at[pl.ds(tbase, BPW)], tidx_v)
    pltpu.sync_copy(ctx_hbm.at[pl.ds(cbase, CPW)], cidx_v)

    iota16 = lax.iota(jnp.int32, 16)

    def sub_batch(sb, carry):
        def prep_t(j, carry2):
            v = tidx_v[pl.ds(sb * SB + j * 16, 16)]
            tg_v[pl.ds(j * 16, 16)] = lax.shift_right_logical(v, 2)
            tq_v[pl.ds(j * 16, 16)] = lax.shift_left(
                lax.bitwise_and(v, 3), 5)
            return carry2

        def prep_c(j, carry2):
            v = cidx_v[pl.ds(sb * SC + j * 16, 16)]
            cg_v[pl.ds(j * 16, 16)] = lax.shift_right_logical(v, 2)
            cq_v[pl.ds(j * 16, 16)] = lax.shift_left(
                lax.bitwise_and(v, 3), 5)
            return carry2

        lax.fori_loop(0, SB // 16, prep_t, 0)
        lax.fori_loop(0, SC // 16, prep_c, 0)

        handles = [pltpu.async_copy(
            ttab_hbm.at[tg_v], trow_v, sem)]
        for k in range(SC // CHUNK):
            handles.append(pltpu.async_copy(
                ctab_hbm.at[cg_v.at[pl.ds(k * CHUNK, CHUNK)]],
                crow_v.at[pl.ds(k * CHUNK, CHUNK)], sem))
        for h in handles:
            h.wait()

        def block(blk, carry2):
            b_vec = blk * 16 + iota16
            tq = tg_q = tq_v[pl.ds(blk * 16, 16)]
            r_vecs = [b_vec * C + c for c in range(C)]
            cqs = [plsc.load_gather(cq_v, [r]) for r in r_vecs]
            acc = [jnp.zeros((16,), jnp.float32) for _ in range(C)]
            for e in range(EMB):
                tv = plsc.load_gather(trow_v, [b_vec, tq + e])
                for c in range(C):
                    cv = plsc.load_gather(crow_v, [r_vecs[c], cqs[c] + e])
                    acc[c] = acc[c] + tv * cv
            for c in range(C):
                plsc.store_scatter(out_v, [sb * SC + r_vecs[c]], acc[c])
            return carry2

        lax.fori_loop(0, SB // 16, block, 0)
        return carry

    lax.fori_loop(0, NSB, sub_batch, 0)

    pltpu.sync_copy(out_v, out_hbm.at[pl.ds(cbase, CPW)])


def kernel(tgt, ctx, target_table, context_table):
    out = _w2v(tgt.reshape(-1), ctx.reshape(-1),
               target_table.reshape(ROWS, PACK * EMB),
               context_table.reshape(ROWS, PACK * EMB))
    return out.reshape(B, C)
